# Initial kernel scaffold; baseline (speedup 1.0000x reference)
import jax
import jax.numpy as jnp
from jax.experimental import pallas as pl


def kernel(x, edge_index, edge_attr, W1, b1, W2, b2, W3, b3):
    raise NotImplementedError("write your pallas kernel here")


# speedup vs baseline: 4.0010x; 4.0010x over previous
import functools

import jax
import jax.numpy as jnp
from jax import lax
from jax.experimental import pallas as pl
from jax.experimental.pallas import tpu as pltpu
from jax.experimental.pallas import tpu_sc as plsc

N, E, D, DE, H = 10000, 320000, 128, 16, 32

NC, NS = 2, 16
NW = NC * NS
CB = 128
JPW = 79
E_PAD = NW * JPW * CB
N_ACC = 10112
RPT = N_ACC // NS
CW = 16


def _nodeproj_body(x_ref, wa_ref, wb_ref, b1_ref, ps_ref, pd_ref):
    x = x_ref[...]
    ps_ref[...] = jnp.dot(x, wa_ref[...], preferred_element_type=jnp.float32)
    pd_ref[...] = jnp.dot(x, wb_ref[...], preferred_element_type=jnp.float32) + b1_ref[...]


def _node_projections(x_pad, w1aT, w1bT, b1):
    blk = N_ACC // 4
    return pl.pallas_call(
        _nodeproj_body,
        grid=(4,),
        in_specs=[
            pl.BlockSpec((blk, D), lambda i: (i, 0)),
            pl.BlockSpec((D, H), lambda i: (0, 0)),
            pl.BlockSpec((D, H), lambda i: (0, 0)),
            pl.BlockSpec((H,), lambda i: (0,)),
        ],
        out_specs=[
            pl.BlockSpec((blk, H), lambda i: (i, 0)),
            pl.BlockSpec((blk, H), lambda i: (i, 0)),
        ],
        out_shape=[
            jax.ShapeDtypeStruct((N_ACC, H), jnp.float32),
            jax.ShapeDtypeStruct((N_ACC, H), jnp.float32),
        ],
    )(x_pad, w1aT, w1bT, b1)


def _attrproj_body(a_ref, w_ref, out_ref):
    out_ref[...] = jnp.dot(a_ref[...], w_ref[...], preferred_element_type=jnp.float32)


def _attr_projection(edge_attr_pad, w1cT):
    blk = 4096
    return pl.pallas_call(
        _attrproj_body,
        grid=(E_PAD // blk,),
        in_specs=[
            pl.BlockSpec((blk, DE), lambda i: (i, 0)),
            pl.BlockSpec((DE, H), lambda i: (0, 0)),
        ],
        out_specs=pl.BlockSpec((blk, H), lambda i: (i, 0)),
        out_shape=jax.ShapeDtypeStruct((E_PAD, H), jnp.float32),
    )(edge_attr_pad, w1cT)


def _edge_body(ps_hbm, pd_hbm, a_hbm, src_hbm, dst_hbm, zacc_hbm, zcnt_hbm, ones_hbm,
               acc_out, cnt_out,
               idx_s, idx_d, rows_s, rows_d, a_v, ones_v, acc_sh, cnt_sh, sem0, sem1):
    cid = lax.axis_index("c")
    sid = lax.axis_index("s")
    wid = sid * NC + cid
    base = sid * RPT

    pltpu.sync_copy(zacc_hbm, acc_sh.at[pl.ds(base, RPT)])
    pltpu.sync_copy(zcnt_hbm, cnt_sh.at[pl.ds(base, RPT)])
    pltpu.sync_copy(ones_hbm, ones_v)
    plsc.subcore_barrier()

    def chunk_body(j, carry):
        c = wid * JPW + j
        pltpu.sync_copy(src_hbm.at[c], idx_s)
        pltpu.sync_copy(dst_hbm.at[c], idx_d)
        g1 = pltpu.async_copy(ps_hbm.at[idx_s], rows_s, sem0)
        g2 = pltpu.async_copy(pd_hbm.at[idx_d], rows_d, sem1)
        pltpu.sync_copy(a_hbm.at[c], a_v)
        g1.wait()
        g2.wait()

        def row_body(i, carry2):
            for half in range(H // 16):
                sl = pl.ds(half * 16, 16)
                v = rows_s[i, sl] + rows_d[i, sl] + a_v[i, sl]
                a_v[i, sl] = jnp.maximum(v, 0.0)
            return carry2

        lax.fori_loop(0, CB, row_body, 0, unroll=2)

        pltpu.sync_copy(a_v, acc_sh.at[idx_d], add=True)
        pltpu.sync_copy(ones_v, cnt_sh.at[idx_d], add=True)
        return carry

    lax.fori_loop(0, JPW, chunk_body, 0)

    plsc.subcore_barrier()
    pltpu.sync_copy(acc_sh.at[pl.ds(base, RPT)], acc_out.at[cid, pl.ds(base, RPT)])
    pltpu.sync_copy(cnt_sh.at[pl.ds(base, RPT)], cnt_out.at[cid, pl.ds(base, RPT)])


_edge_kernel = pl.kernel(
    _edge_body,
    out_type=[
        jax.ShapeDtypeStruct((NC, N_ACC, H), jnp.float32),
        jax.ShapeDtypeStruct((NC, N_ACC, CW), jnp.float32),
    ],
    mesh=plsc.VectorSubcoreMesh(core_axis_name="c", subcore_axis_name="s"),
    scratch_types=[
        pltpu.VMEM((CB,), jnp.int32),
        pltpu.VMEM((CB,), jnp.int32),
        pltpu.VMEM((CB, H), jnp.float32),
        pltpu.VMEM((CB, H), jnp.float32),
        pltpu.VMEM((CB, H), jnp.float32),
        pltpu.VMEM((CB, CW), jnp.float32),
        pltpu.VMEM_SHARED((N_ACC, H), jnp.float32),
        pltpu.VMEM_SHARED((N_ACC, CW), jnp.float32),
        pltpu.SemaphoreType.DMA,
        pltpu.SemaphoreType.DMA,
    ],
    compiler_params=pltpu.CompilerParams(use_tc_tiling_on_sc=False),
)


def _post_body(x_ref, a0_ref, a1_ref, c0_ref, c1_ref, w2T_ref, b2_ref,
               w3aT_ref, w3bT_ref, b3_ref, out_ref):
    x = x_ref[...]
    hs = a0_ref[0] + a1_ref[0]
    cnt = c0_ref[0][:, 0:1] + c1_ref[0][:, 0:1]
    agg = (jnp.dot(hs, w2T_ref[...], preferred_element_type=jnp.float32)
           + cnt * b2_ref[...]) / (cnt + 1e-6)
    xn = (jnp.dot(x, w3aT_ref[...], preferred_element_type=jnp.float32)
          + jnp.dot(agg, w3bT_ref[...], preferred_element_type=jnp.float32)
          + b3_ref[...])
    out_ref[...] = x + jnp.maximum(xn, 0.0)


def _post_update(x, accs, cnts, w2T, b2, w3aT, w3bT, b3):
    blk = 1000
    return pl.pallas_call(
        _post_body,
        grid=(N // blk,),
        in_specs=[
            pl.BlockSpec((blk, D), lambda i: (i, 0)),
            pl.BlockSpec((1, blk, H), lambda i: (0, i, 0)),
            pl.BlockSpec((1, blk, H), lambda i: (1, i, 0)),
            pl.BlockSpec((1, blk, CW), lambda i: (0, i, 0)),
            pl.BlockSpec((1, blk, CW), lambda i: (1, i, 0)),
            pl.BlockSpec((H, H), lambda i: (0, 0)),
            pl.BlockSpec((H,), lambda i: (0,)),
            pl.BlockSpec((D, D), lambda i: (0, 0)),
            pl.BlockSpec((H, D), lambda i: (0, 0)),
            pl.BlockSpec((D,), lambda i: (0,)),
        ],
        out_specs=pl.BlockSpec((blk, D), lambda i: (i, 0)),
        out_shape=jax.ShapeDtypeStruct((N, D), jnp.float32),
    )(x, accs, accs, cnts, cnts, w2T, b2, w3aT, w3bT, b3)


def kernel(x, edge_index, edge_attr, W1, b1, W2, b2, W3, b3):
    w1aT = W1[:, :D].T
    w1bT = W1[:, D:2 * D].T
    w1cT = W1[:, 2 * D:].T
    w2T = W2.T
    w3aT = W3[:, :D].T
    w3bT = W3[:, D:].T

    x_pad = jnp.pad(x, ((0, N_ACC - N), (0, 0)))
    src = jnp.pad(edge_index[0], (0, E_PAD - E)).reshape(NW * JPW, CB)
    dst = jnp.pad(edge_index[1], (0, E_PAD - E), constant_values=N).reshape(NW * JPW, CB)
    attr_pad = jnp.pad(edge_attr, ((0, E_PAD - E), (0, 0)))

    ps, pd = _node_projections(x_pad, w1aT, w1bT, b1)
    a = _attr_projection(attr_pad, w1cT)

    zacc = jnp.zeros((RPT, H), jnp.float32)
    zcnt = jnp.zeros((RPT, CW), jnp.float32)
    ones = jnp.ones((CB, CW), jnp.float32)

    accs, cnts = _edge_kernel(ps, pd, a.reshape(NW * JPW, CB, H), src, dst,
                              zacc, zcnt, ones)

    return _post_update(x, accs, cnts, w2T, b2, w3aT, w3bT, b3)


# speedup vs baseline: 7.3417x; 1.8350x over previous
import functools

import jax
import jax.numpy as jnp
from jax import lax
from jax.experimental import pallas as pl
from jax.experimental.pallas import tpu as pltpu
from jax.experimental.pallas import tpu_sc as plsc

N, E, D, DE, H = 10000, 320000, 128, 16, 32

NC, NS = 2, 16
NW = NC * NS
CB = 128
JPW = 80
E_PAD = NW * JPW * CB
N_ACC = 10112
RPT = N_ACC // NS
AW = H + 8


def _nodeproj_body(x_ref, wa_ref, wb_ref, b1_ref, ps_ref, pd_ref):
    x = x_ref[...]
    ps_ref[...] = jnp.dot(x, wa_ref[...], preferred_element_type=jnp.float32)
    pd_ref[...] = jnp.dot(x, wb_ref[...], preferred_element_type=jnp.float32) + b1_ref[...]


def _node_projections(x_pad, w1aT, w1bT, b1):
    blk = N_ACC // 4
    return pl.pallas_call(
        _nodeproj_body,
        grid=(4,),
        in_specs=[
            pl.BlockSpec((blk, D), lambda i: (i, 0)),
            pl.BlockSpec((D, H), lambda i: (0, 0)),
            pl.BlockSpec((D, H), lambda i: (0, 0)),
            pl.BlockSpec((H,), lambda i: (0,)),
        ],
        out_specs=[
            pl.BlockSpec((blk, H), lambda i: (i, 0)),
            pl.BlockSpec((blk, H), lambda i: (i, 0)),
        ],
        out_shape=[
            jax.ShapeDtypeStruct((N_ACC, H), jnp.float32),
            jax.ShapeDtypeStruct((N_ACC, H), jnp.float32),
        ],
    )(x_pad, w1aT, w1bT, b1)


def _attrproj_body(a_ref, w_ref, out_ref):
    out_ref[...] = jnp.dot(a_ref[...], w_ref[...], preferred_element_type=jnp.float32)


def _attr_projection(edge_attr8, w_bd):
    blk = 4096
    e8_pad = E_PAD // 8
    return pl.pallas_call(
        _attrproj_body,
        grid=(e8_pad // blk,),
        in_specs=[
            pl.BlockSpec((blk, 8 * DE), lambda i: (i, 0)),
            pl.BlockSpec((8 * DE, 8 * H), lambda i: (0, 0)),
        ],
        out_specs=pl.BlockSpec((blk, 8 * H), lambda i: (i, 0)),
        out_shape=jax.ShapeDtypeStruct((e8_pad, 8 * H), jnp.float32),
    )(edge_attr8, w_bd)


def _edge_body(ps_hbm, pd_hbm, a_hbm, src_hbm, dst_hbm, zacc_hbm, hinit_hbm,
               acc_out,
               idx_s, idx_d, rows_s0, rows_d0, a_v0, h_v0, rows_s1, rows_d1,
               a_v1, h_v1, acc_sh,
               gs0, gd0, ga0, ss0, gs1, gd1, ga1, ss1):
    cid = lax.axis_index("c")
    sid = lax.axis_index("s")
    wid = sid * NC + cid
    base = sid * RPT

    pltpu.sync_copy(zacc_hbm, acc_sh.at[pl.ds(base, RPT)])
    pltpu.sync_copy(hinit_hbm, h_v0)
    pltpu.sync_copy(hinit_hbm, h_v1)
    pltpu.sync_copy(src_hbm.at[wid], idx_s)
    pltpu.sync_copy(dst_hbm.at[wid], idx_d)
    plsc.subcore_barrier()

    bufs = ((rows_s0, rows_d0, a_v0, h_v0, gs0, gd0, ga0, ss0),
            (rows_s1, rows_d1, a_v1, h_v1, gs1, gd1, ga1, ss1))

    def start_gathers(jj, b):
        rs, rd, av = bufs[b][0], bufs[b][1], bufs[b][2]
        pltpu.async_copy(ps_hbm.at[idx_s.at[jj]], rs, bufs[b][4])
        pltpu.async_copy(pd_hbm.at[idx_d.at[jj]], rd, bufs[b][5])
        pltpu.async_copy(a_hbm.at[wid, jj], av, bufs[b][6])

    start_gathers(0, 0)
    start_gathers(1, 1)

    def pair_body(i, carry):
        for b in range(2):
            jj = 2 * i + b
            rs, rd, av, hv, gs, gd, ga, ss = bufs[b]
            pltpu.make_async_copy(ps_hbm.at[idx_s.at[jj]], rs, gs).wait()
            pltpu.make_async_copy(pd_hbm.at[idx_d.at[jj]], rd, gd).wait()
            pltpu.make_async_copy(a_hbm.at[wid, jj], av, ga).wait()

            @pl.when(i > 0)
            def _():
                pltpu.make_async_copy(hv, acc_sh.at[idx_d.at[jj]], ss).wait()

            def row_body(r, carry2):
                for half in range(H // 16):
                    sl = pl.ds(half * 16, 16)
                    v = rs[r, sl] + rd[r, sl] + av[r, sl]
                    hv[r, sl] = jnp.maximum(v, 0.0)
                return carry2

            lax.fori_loop(0, CB, row_body, 0, unroll=4)

            pltpu.async_copy(hv, acc_sh.at[idx_d.at[jj]], ss, add=True)

            @pl.when(i < JPW // 2 - 1)
            def _():
                start_gathers(jj + 2, b)
        return carry

    lax.fori_loop(0, JPW // 2, pair_body, 0)

    pltpu.make_async_copy(h_v0, acc_sh.at[idx_d.at[JPW - 2]], ss0).wait()
    pltpu.make_async_copy(h_v1, acc_sh.at[idx_d.at[JPW - 1]], ss1).wait()
    plsc.subcore_barrier()
    pltpu.sync_copy(acc_sh.at[pl.ds(base, RPT)], acc_out.at[cid, pl.ds(base, RPT)])


_edge_kernel = pl.kernel(
    _edge_body,
    out_type=jax.ShapeDtypeStruct((NC, N_ACC, AW), jnp.float32),
    mesh=plsc.VectorSubcoreMesh(core_axis_name="c", subcore_axis_name="s"),
    scratch_types=[
        pltpu.VMEM((JPW, CB), jnp.int32),
        pltpu.VMEM((JPW, CB), jnp.int32),
        pltpu.VMEM((CB, H), jnp.float32),
        pltpu.VMEM((CB, H), jnp.float32),
        pltpu.VMEM((CB, H), jnp.float32),
        pltpu.VMEM((CB, AW), jnp.float32),
        pltpu.VMEM((CB, H), jnp.float32),
        pltpu.VMEM((CB, H), jnp.float32),
        pltpu.VMEM((CB, H), jnp.float32),
        pltpu.VMEM((CB, AW), jnp.float32),
        pltpu.VMEM_SHARED((N_ACC, AW), jnp.float32),
        pltpu.SemaphoreType.DMA,
        pltpu.SemaphoreType.DMA,
        pltpu.SemaphoreType.DMA,
        pltpu.SemaphoreType.DMA,
        pltpu.SemaphoreType.DMA,
        pltpu.SemaphoreType.DMA,
        pltpu.SemaphoreType.DMA,
        pltpu.SemaphoreType.DMA,
    ],
    compiler_params=pltpu.CompilerParams(use_tc_tiling_on_sc=False),
)


def _post_body(x_ref, a0_ref, a1_ref, w2T_ref, b2_ref,
               w3aT_ref, w3bT_ref, b3_ref, out_ref):
    x = x_ref[...]
    acc = a0_ref[0] + a1_ref[0]
    hs = acc[:, :H]
    cnt = acc[:, H:H + 1]
    agg = (jnp.dot(hs, w2T_ref[...], preferred_element_type=jnp.float32)
           + cnt * b2_ref[...]) / (cnt + 1e-6)
    xn = (jnp.dot(x, w3aT_ref[...], preferred_element_type=jnp.float32)
          + jnp.dot(agg, w3bT_ref[...], preferred_element_type=jnp.float32)
          + b3_ref[...])
    out_ref[...] = x + jnp.maximum(xn, 0.0)


def _post_update(x, accs, w2T, b2, w3aT, w3bT, b3):
    blk = 1000
    return pl.pallas_call(
        _post_body,
        grid=(N // blk,),
        in_specs=[
            pl.BlockSpec((blk, D), lambda i: (i, 0)),
            pl.BlockSpec((1, blk, AW), lambda i: (0, i, 0)),
            pl.BlockSpec((1, blk, AW), lambda i: (1, i, 0)),
            pl.BlockSpec((H, H), lambda i: (0, 0)),
            pl.BlockSpec((H,), lambda i: (0,)),
            pl.BlockSpec((D, D), lambda i: (0, 0)),
            pl.BlockSpec((H, D), lambda i: (0, 0)),
            pl.BlockSpec((D,), lambda i: (0,)),
        ],
        out_specs=pl.BlockSpec((blk, D), lambda i: (i, 0)),
        out_shape=jax.ShapeDtypeStruct((N, D), jnp.float32),
    )(x, accs, accs, w2T, b2, w3aT, w3bT, b3)


def kernel(x, edge_index, edge_attr, W1, b1, W2, b2, W3, b3):
    w1aT = W1[:, :D].T
    w1bT = W1[:, D:2 * D].T
    w1cT = W1[:, 2 * D:].T
    w2T = W2.T
    w3aT = W3[:, :D].T
    w3bT = W3[:, D:].T

    x_pad = jnp.pad(x, ((0, N_ACC - N), (0, 0)))
    src = jnp.pad(edge_index[0], (0, E_PAD - E)).reshape(NW, JPW, CB)
    dst = jnp.pad(edge_index[1], (0, E_PAD - E), constant_values=N).reshape(NW, JPW, CB)

    ps, pd = _node_projections(x_pad, w1aT, w1bT, b1)

    w_bd = jnp.kron(jnp.eye(8, dtype=jnp.float32), w1cT)
    a8 = _attr_projection(edge_attr.reshape(E // 8, 8 * DE), w_bd)
    a = a8.reshape(NW, JPW, CB, H)

    zacc = jnp.zeros((RPT, AW), jnp.float32)
    hinit = jnp.zeros((CB, AW), jnp.float32).at[:, H].set(1.0)

    accs = _edge_kernel(ps, pd, a, src, dst, zacc, hinit)

    return _post_update(x, accs, w2T, b2, w3aT, w3bT, b3)
